# baseline (speedup 1.0000x reference)
import jax
import jax.numpy as jnp
from jax import lax
from jax.experimental import pallas as pl
from jax.experimental.pallas import tpu as pltpu
from jax.experimental.pallas import tpu_sc as plsc

_NUM_CORES = 2
_NUM_SUBCORES = 16
_NBUF = 2
_CHUNK = 320


def _sc_gather(W, idx_flat, n, D):
    mesh = plsc.VectorSubcoreMesh(core_axis_name="c", subcore_axis_name="s")
    nw = _NUM_CORES * _NUM_SUBCORES
    per_w = n // nw
    chunks = per_w // _CHUNK
    iters = chunks // _NBUF

    @pl.kernel(
        out_type=jax.ShapeDtypeStruct((n, D), W.dtype),
        mesh=mesh,
        scratch_types=[
            pltpu.VMEM((per_w,), jnp.int32),
            pltpu.VMEM((_NBUF, _CHUNK, D), W.dtype),
            pltpu.SemaphoreType.DMA((_NBUF,)),
            pltpu.SemaphoreType.DMA((_NBUF,)),
            pltpu.SemaphoreType.DMA,
        ],
    )
    def gather_kernel(w_hbm, i_hbm, o_hbm, idx_v, buf, gsem, wsem, isem):
        wid = lax.axis_index("s") * _NUM_CORES + lax.axis_index("c")
        base = wid * per_w
        pltpu.async_copy(i_hbm.at[pl.ds(base, per_w)], idx_v, isem).wait()

        def gather_chunk(c, b):
            src = w_hbm.at[idx_v.at[pl.ds(c * _CHUNK, _CHUNK)]]
            pltpu.make_async_copy(src, buf.at[b], gsem.at[b]).start()

        def wait_gather(c, b):
            src = w_hbm.at[idx_v.at[pl.ds(c * _CHUNK, _CHUNK)]]
            pltpu.make_async_copy(src, buf.at[b], gsem.at[b]).wait()

        def write_chunk(c, b):
            dst = o_hbm.at[pl.ds(base + c * _CHUNK, _CHUNK)]
            pltpu.make_async_copy(buf.at[b], dst, wsem.at[b]).start()

        def wait_write(c, b):
            dst = o_hbm.at[pl.ds(base + c * _CHUNK, _CHUNK)]
            pltpu.make_async_copy(buf.at[b], dst, wsem.at[b]).wait()

        for b in range(_NBUF):
            gather_chunk(jnp.int32(b), b)

        @pl.loop(0, iters - 1)
        def _(k):
            c0 = k * _NBUF
            for b in range(_NBUF):
                wait_gather(c0 + b, b)
                write_chunk(c0 + b, b)
            for b in range(_NBUF):
                wait_write(c0 + b, b)
                gather_chunk(c0 + _NBUF + b, b)

        cl = jnp.int32((iters - 1) * _NBUF)
        for b in range(_NBUF):
            wait_gather(cl + b, b)
            write_chunk(cl + b, b)
        for b in range(_NBUF):
            wait_write(cl + b, b)

    return gather_kernel(W, idx_flat)


def kernel(x, W):
    B, L = x.shape
    V, D = W.shape
    n = B * L
    idx_flat = x.reshape(n).astype(jnp.int32)
    out = _sc_gather(W, idx_flat, n, D)
    return out.reshape(B, L, D)


# speedup vs baseline: 1.0691x; 1.0691x over previous
import jax
import jax.numpy as jnp
from jax import lax
from jax.experimental import pallas as pl
from jax.experimental.pallas import tpu as pltpu
from jax.experimental.pallas import tpu_sc as plsc

_NUM_CORES = 2
_NUM_SUBCORES = 16
_NBUF = 10
_CHUNK = 64


def _sc_gather(W, idx_flat, n, D):
    mesh = plsc.VectorSubcoreMesh(core_axis_name="c", subcore_axis_name="s")
    nw = _NUM_CORES * _NUM_SUBCORES
    per_w = n // nw
    chunks = per_w // _CHUNK
    iters = chunks // _NBUF

    @pl.kernel(
        out_type=jax.ShapeDtypeStruct((n, D), W.dtype),
        mesh=mesh,
        scratch_types=[
            pltpu.VMEM((per_w,), jnp.int32),
            pltpu.VMEM((_NBUF, _CHUNK, D), W.dtype),
            pltpu.SemaphoreType.DMA((_NBUF,)),
            pltpu.SemaphoreType.DMA((_NBUF,)),
            pltpu.SemaphoreType.DMA,
        ],
    )
    def gather_kernel(w_hbm, i_hbm, o_hbm, idx_v, buf, gsem, wsem, isem):
        wid = lax.axis_index("s") * _NUM_CORES + lax.axis_index("c")
        base = wid * per_w
        pltpu.async_copy(i_hbm.at[pl.ds(base, per_w)], idx_v, isem).wait()

        def gather_chunk(c, b):
            src = w_hbm.at[idx_v.at[pl.ds(c * _CHUNK, _CHUNK)]]
            pltpu.make_async_copy(src, buf.at[b], gsem.at[b]).start()

        def wait_gather(c, b):
            src = w_hbm.at[idx_v.at[pl.ds(c * _CHUNK, _CHUNK)]]
            pltpu.make_async_copy(src, buf.at[b], gsem.at[b]).wait()

        def write_chunk(c, b):
            dst = o_hbm.at[pl.ds(base + c * _CHUNK, _CHUNK)]
            pltpu.make_async_copy(buf.at[b], dst, wsem.at[b]).start()

        def wait_write(c, b):
            dst = o_hbm.at[pl.ds(base + c * _CHUNK, _CHUNK)]
            pltpu.make_async_copy(buf.at[b], dst, wsem.at[b]).wait()

        for b in range(_NBUF):
            gather_chunk(jnp.int32(b), b)

        @pl.loop(0, iters - 1)
        def _(k):
            c0 = k * _NBUF
            for b in range(_NBUF):
                wait_gather(c0 + b, b)
                write_chunk(c0 + b, b)
            for b in range(_NBUF):
                wait_write(c0 + b, b)
                gather_chunk(c0 + _NBUF + b, b)

        cl = jnp.int32((iters - 1) * _NBUF)
        for b in range(_NBUF):
            wait_gather(cl + b, b)
            write_chunk(cl + b, b)
        for b in range(_NBUF):
            wait_write(cl + b, b)

    return gather_kernel(W, idx_flat)


def kernel(x, W):
    B, L = x.shape
    V, D = W.shape
    n = B * L
    idx_flat = x.reshape(n).astype(jnp.int32)
    out = _sc_gather(W, idx_flat, n, D)
    return out.reshape(B, L, D)
